# Initial kernel scaffold; baseline (speedup 1.0000x reference)
import jax
import jax.numpy as jnp
from jax.experimental import pallas as pl


def kernel(ids, weight):
    raise NotImplementedError("write your pallas kernel here")


# speedup vs baseline: 4.2408x; 4.2408x over previous
import functools

import jax
import jax.numpy as jnp
from jax import lax
from jax.experimental import pallas as pl
from jax.experimental.pallas import tpu as pltpu
from jax.experimental.pallas import tpu_sc as plsc

_MAX_NORM = 1.0 - 1e-5


def _project_block(w_ref, o_ref):
    x = w_ref[...]
    norm = jnp.sqrt(jnp.sum(x * x, axis=-1, keepdims=True))
    safe = jnp.maximum(norm, 1e-15)
    proj = x / safe * _MAX_NORM
    o_ref[...] = jnp.where(norm > _MAX_NORM, proj, x)


def _project_table(weight):
    n, d = weight.shape
    blk = 8000
    return pl.pallas_call(
        _project_block,
        grid=(n // blk,),
        in_specs=[pl.BlockSpec((blk, d), lambda i: (i, 0))],
        out_specs=pl.BlockSpec((blk, d), lambda i: (i, 0)),
        out_shape=jax.ShapeDtypeStruct((n, d), weight.dtype),
    )(weight)


_NC, _NS = 2, 16
_NW = _NC * _NS
_D = 32
_B = 16384 * 200
_SUB = 128
_CHUNK = 1024
_NSUB = _CHUNK // _SUB
_BPW = _B // _NW
_NCHUNK = _BPW // _CHUNK


def _gather_kernel(table, idx2d, out, idx_v, rows_v, idx_sem, g_sem, s_sem):
    wid = lax.axis_index("s") * _NC + lax.axis_index("c")
    row0 = wid * _BPW
    irow0 = row0 // _SUB

    def idx_copy(g, b):
        off = pl.multiple_of(irow0 + g * _NSUB, _NSUB)
        return pltpu.make_async_copy(
            idx2d.at[pl.ds(off, _NSUB)], idx_v.at[b],
            idx_sem.at[b])

    def gather_copy(b, j):
        return pltpu.make_async_copy(
            table.at[idx_v.at[b, j]],
            rows_v.at[b, pl.ds(j * _SUB, _SUB)],
            g_sem.at[b])

    def scatter_copy(g, b):
        return pltpu.make_async_copy(
            rows_v.at[b], out.at[pl.ds(row0 + g * _CHUNK, _CHUNK)],
            s_sem.at[b])

    for b in range(2):
        idx_copy(b, b).start()

    def body(i, carry):
        gg = i * 2
        for b in range(2):
            g = gg + b
            idx_copy(g, b).wait()

            @pl.when(g >= 2)
            def _wait_prev_scatter():
                scatter_copy(g - 2, b).wait()

            for j in range(_NSUB):
                gather_copy(b, j).start()
            for j in range(_NSUB):
                gather_copy(b, j).wait()

            @pl.when(g + 2 < _NCHUNK)
            def _prefetch_ids():
                idx_copy(g + 2, b).start()

            scatter_copy(g, b).start()
        return carry

    lax.fori_loop(0, _NCHUNK // 2, body, 0)

    for b in range(2):
        scatter_copy(_NCHUNK - 2 + b, b).wait()


def _sc_gather(ptable, ids2d):
    mesh = plsc.VectorSubcoreMesh(core_axis_name="c", subcore_axis_name="s")
    f = pl.kernel(
        _gather_kernel,
        mesh=mesh,
        out_type=jax.ShapeDtypeStruct((_B, _D), jnp.float32),
        scratch_types=[
            pltpu.VMEM((2, _NSUB, _SUB), jnp.int32),
            pltpu.VMEM((2, _CHUNK, _D), jnp.float32),
            pltpu.SemaphoreType.DMA((2,)),
            pltpu.SemaphoreType.DMA((2,)),
            pltpu.SemaphoreType.DMA((2,)),
        ],
        compiler_params=pltpu.CompilerParams(use_tc_tiling_on_sc=False),
    )
    return f(ptable, ids2d)


def kernel(ids, weight):
    ptable = _project_table(weight)
    ids2d = ids.reshape(_B // _SUB, _SUB)
    out = _sc_gather(ptable, ids2d)
    return out.reshape(ids.shape + (weight.shape[1],))


# speedup vs baseline: 4.6475x; 1.0959x over previous
import functools

import jax
import jax.numpy as jnp
from jax import lax
from jax.experimental import pallas as pl
from jax.experimental.pallas import tpu as pltpu
from jax.experimental.pallas import tpu_sc as plsc

_MAX_NORM = 1.0 - 1e-5


def _project_block_t(w_ref, o_ref):
    x = w_ref[...]
    norm = jnp.sqrt(jnp.sum(x * x, axis=0, keepdims=True))
    safe = jnp.maximum(norm, 1e-15)
    proj = x / safe * _MAX_NORM
    o_ref[...] = jnp.where(norm > _MAX_NORM, proj, x)


def _project_table_t(weight_t):
    d, n = weight_t.shape
    blk = 12800
    return pl.pallas_call(
        _project_block_t,
        grid=(pl.cdiv(n, blk),),
        in_specs=[pl.BlockSpec((d, blk), lambda i: (0, i))],
        out_specs=pl.BlockSpec((d, blk), lambda i: (0, i)),
        out_shape=jax.ShapeDtypeStruct((d, n), weight_t.dtype),
    )(weight_t)


_NC, _NS = 2, 16
_NW = _NC * _NS
_D = 32
_B = 16384 * 200
_SUB = 128
_CHUNK = 1024
_NSUB = _CHUNK // _SUB
_BPW = _B // _NW
_NCHUNK = _BPW // _CHUNK


def _gather_kernel(table, idx2d, out, idx_v, rows_v, idx_sem, g_sem, s_sem):
    wid = lax.axis_index("s") * _NC + lax.axis_index("c")
    row0 = wid * _BPW
    irow0 = row0 // _SUB

    def idx_copy(g, b):
        off = pl.multiple_of(irow0 + g * _NSUB, _NSUB)
        return pltpu.make_async_copy(
            idx2d.at[pl.ds(off, _NSUB)], idx_v.at[b],
            idx_sem.at[b])

    def gather_copy(b, j):
        return pltpu.make_async_copy(
            table.at[idx_v.at[b, j]],
            rows_v.at[b, pl.ds(j * _SUB, _SUB)],
            g_sem.at[b])

    def scatter_copy(g, b):
        return pltpu.make_async_copy(
            rows_v.at[b], out.at[pl.ds(row0 + g * _CHUNK, _CHUNK)],
            s_sem.at[b])

    for b in range(2):
        idx_copy(b, b).start()

    def body(i, carry):
        gg = i * 2
        for b in range(2):
            g = gg + b
            idx_copy(g, b).wait()

            @pl.when(g >= 2)
            def _wait_prev_scatter():
                scatter_copy(g - 2, b).wait()

            for j in range(_NSUB):
                gather_copy(b, j).start()
            for j in range(_NSUB):
                gather_copy(b, j).wait()

            @pl.when(g + 2 < _NCHUNK)
            def _prefetch_ids():
                idx_copy(g + 2, b).start()

            scatter_copy(g, b).start()
        return carry

    lax.fori_loop(0, _NCHUNK // 2, body, 0)

    for b in range(2):
        scatter_copy(_NCHUNK - 2 + b, b).wait()


def _sc_gather(ptable, ids2d):
    mesh = plsc.VectorSubcoreMesh(core_axis_name="c", subcore_axis_name="s")
    f = pl.kernel(
        _gather_kernel,
        mesh=mesh,
        out_type=jax.ShapeDtypeStruct((_B, _D), jnp.float32),
        scratch_types=[
            pltpu.VMEM((2, _NSUB, _SUB), jnp.int32),
            pltpu.VMEM((2, _CHUNK, _D), jnp.float32),
            pltpu.SemaphoreType.DMA((2,)),
            pltpu.SemaphoreType.DMA((2,)),
            pltpu.SemaphoreType.DMA((2,)),
        ],
        compiler_params=pltpu.CompilerParams(use_tc_tiling_on_sc=False),
    )
    return f(ptable, ids2d)


def kernel(ids, weight):
    n, d = weight.shape
    ptable_t = _project_table_t(weight.T)
    ptable = ptable_t.T.reshape(n // 4, 4 * d).reshape(n, d)
    ids2d = ids.reshape(_B // _SUB, _SUB)
    out = _sc_gather(ptable, ids2d)
    return out.reshape(ids.shape + (d,))


# speedup vs baseline: 5.6417x; 1.2139x over previous
import functools

import jax
import jax.numpy as jnp
from jax import lax
from jax.experimental import pallas as pl
from jax.experimental.pallas import tpu as pltpu
from jax.experimental.pallas import tpu_sc as plsc

_MAX_NORM = 1.0 - 1e-5

_N = 1000000
_D = 32
_BK = 8192
_NBLK = 1007616 // _BK
_NPAD = _NBLK * _BK


def _project_pack_block(w_ref, o_ref):
    x = w_ref[...]
    norm = jnp.sqrt(jnp.sum(x * x, axis=0, keepdims=True))
    safe = jnp.maximum(norm, 1e-15)
    proj = x / safe * _MAX_NORM
    y = jnp.where(norm > _MAX_NORM, proj, x)
    for c in range(4):
        z = y[:, 2048 * c:2048 * (c + 1)].T
        o_ref[:, :, 32 * c:32 * (c + 1)] = z.reshape(256, 8, 32)


def _project_table_packed(weight_t):
    return pl.pallas_call(
        _project_pack_block,
        grid=(_NBLK,),
        in_specs=[pl.BlockSpec((_D, _BK), lambda i: (0, i))],
        out_specs=pl.BlockSpec((256, 8, 128), lambda i: (i, 0, 0)),
        out_shape=jax.ShapeDtypeStruct((_NPAD // 32, 8, 128), jnp.float32),
    )(weight_t)


_NC, _NS = 2, 16
_NW = _NC * _NS
_B = 16384 * 200
_SUB = 128
_CHUNK = 1024
_NSUB = _CHUNK // _SUB
_BPW = _B // _NW
_NCHUNK = _BPW // _CHUNK


def _gather_kernel(table, idx2d, out, idx_v, idxq_v, rows_v,
                   idx_sem, g_sem, s_sem):
    wid = lax.axis_index("s") * _NC + lax.axis_index("c")
    row0 = wid * _BPW
    irow0 = row0 // _SUB

    def idx_copy(g, b):
        off = pl.multiple_of(irow0 + g * _NSUB, _NSUB)
        return pltpu.make_async_copy(
            idx2d.at[pl.ds(off, _NSUB)], idx_v.at[b],
            idx_sem.at[b])

    def gather_copy(b, j):
        return pltpu.make_async_copy(
            table.at[idxq_v.at[b, j]],
            rows_v.at[b, pl.ds(j * _SUB, _SUB)],
            g_sem.at[b])

    def scatter_copy(g, b):
        return pltpu.make_async_copy(
            rows_v.at[b], out.at[pl.ds(row0 + g * _CHUNK, _CHUNK)],
            s_sem.at[b])

    def transform(b):
        for j in range(_NSUB):
            for k in range(8):
                v = idx_v[b, j, pl.ds(16 * k, 16)]
                m = jnp.bitwise_and(v, 2047)
                c = jnp.bitwise_and(jnp.right_shift(v, 11), 3)
                q = jnp.bitwise_or(
                    jnp.bitwise_or(jnp.bitwise_and(v, ~8191),
                                   jnp.left_shift(jnp.right_shift(m, 3), 5)),
                    jnp.bitwise_or(
                        jnp.left_shift(jnp.bitwise_and(m, 7), 2), c))
                idxq_v[b, j, pl.ds(16 * k, 16)] = q

    for b in range(2):
        idx_copy(b, b).start()

    def body(i, carry):
        gg = i * 2
        for b in range(2):
            g = gg + b
            idx_copy(g, b).wait()
            transform(b)

            @pl.when(g + 2 < _NCHUNK)
            def _prefetch_ids():
                idx_copy(g + 2, b).start()

            @pl.when(g >= 2)
            def _wait_prev_scatter():
                scatter_copy(g - 2, b).wait()

            for j in range(_NSUB):
                gather_copy(b, j).start()
            for j in range(_NSUB):
                gather_copy(b, j).wait()

            scatter_copy(g, b).start()
        return carry

    lax.fori_loop(0, _NCHUNK // 2, body, 0)

    for b in range(2):
        scatter_copy(_NCHUNK - 2 + b, b).wait()


def _sc_gather(ptable_rows, ids2d):
    mesh = plsc.VectorSubcoreMesh(core_axis_name="c", subcore_axis_name="s")
    f = pl.kernel(
        _gather_kernel,
        mesh=mesh,
        out_type=jax.ShapeDtypeStruct((_B, _D), jnp.float32),
        scratch_types=[
            pltpu.VMEM((2, _NSUB, _SUB), jnp.int32),
            pltpu.VMEM((2, _NSUB, _SUB), jnp.int32),
            pltpu.VMEM((2, _CHUNK, _D), jnp.float32),
            pltpu.SemaphoreType.DMA((2,)),
            pltpu.SemaphoreType.DMA((2,)),
            pltpu.SemaphoreType.DMA((2,)),
        ],
        compiler_params=pltpu.CompilerParams(use_tc_tiling_on_sc=False),
    )
    return f(ptable_rows, ids2d)


def kernel(ids, weight):
    n, d = weight.shape
    packed = _project_table_packed(weight.T)
    ptable_rows = packed.reshape(_NPAD, _D)
    ids2d = ids.reshape(_B // _SUB, _SUB)
    out = _sc_gather(ptable_rows, ids2d)
    return out.reshape(ids.shape + (d,))
